# baseline (speedup 1.0000x reference)
import functools

import jax
import jax.numpy as jnp
from jax import lax
from jax.experimental import pallas as pl
from jax.experimental.pallas import tpu as pltpu
from jax.experimental.pallas import tpu_sc as plsc

B, N, E, BETA = 8, 1024, 2048, 16
NH = 128


def _enc_body(scb_ref, xs_ref, xd_ref, xp_ref,
              e1w1, e1b1, e1w2, e1b2, w1, e2b1c,
              e2w2, e2b2, e4w1, e4b1, e4w2, e4b2, nw, nb,
              ea_ref, h0_ref, m1_ref):
    SCB = jnp.abs(scb_ref[0])
    xs = xs_ref[0]
    xd = xd_ref[0]
    dg = lambda a, b, dn: lax.dot_general(a, b, (dn, ((), ())),
                                          preferred_element_type=jnp.float32)
    sA = SCB @ xs
    sB = SCB @ xd
    h1 = (jnp.maximum(sA @ e1w1[0:2, :] + sB @ e1w1[2:4, :] + e1b1[...], 0.0)
          @ e1w2[...] + e1b2[...])
    baseT = dg(w1[0:64, :], h1, ((0,), (1,))) + e2b1c[...]
    bb = [jnp.broadcast_to(baseT[:, k:k + 1], (NH, NH)) for k in range(BETA)]
    epT = (dg(w1[64:66, :], xs, ((0,), (1,)))
           + dg(w1[66:68, :], xd, ((0,), (1,))))
    g1 = xs @ nw[...] + nb[...]
    Ec = 128
    for c in range(E // Ec):
        sl = slice(c * Ec, (c + 1) * Ec)
        epc = epT[:, sl]
        acc = jnp.maximum(SCB[0:1, sl] * epc + bb[0], 0.0)
        for k in range(1, BETA):
            acc = acc + jnp.maximum(SCB[k:k + 1, sl] * epc + bb[k], 0.0)
        emb = dg(acc, e2w2[...], ((0,), (0,))) + BETA * e2b2[...]
        ea = (jnp.maximum(emb @ e4w1[...] + e4b1[...], 0.0)
              @ e4w2[...] + e4b2[...])
        ea_ref[0, sl, :] = ea
        m1_ref[0, sl, :] = jnp.maximum(g1[sl, :] + ea, 0.0)
    h0_ref[0] = xp_ref[0] @ nw[...] + nb[...]


def _full(shape):
    nd = len(shape)
    return pl.BlockSpec(shape, lambda *_, _n=nd: (0,) * _n)


def _edge_encoder(scb, xs_g, xd_g, xpad, *ws):
    return pl.pallas_call(
        _enc_body,
        grid=(B,),
        in_specs=[
            pl.BlockSpec((1, BETA, E), lambda b: (b, 0, 0)),
            pl.BlockSpec((1, E, 2), lambda b: (b, 0, 0)),
            pl.BlockSpec((1, E, 2), lambda b: (b, 0, 0)),
            pl.BlockSpec((1, N, 2), lambda b: (b, 0, 0)),
        ] + [_full(w.shape) for w in ws],
        out_specs=[pl.BlockSpec((1, E, NH), lambda b: (b, 0, 0)),
                   pl.BlockSpec((1, N, NH), lambda b: (b, 0, 0)),
                   pl.BlockSpec((1, E, NH), lambda b: (b, 0, 0))],
        out_shape=[jax.ShapeDtypeStruct((B, E, NH), jnp.float32),
                   jax.ShapeDtypeStruct((B, N, NH), jnp.float32),
                   jax.ShapeDtypeStruct((B, E, NH), jnp.float32)],
    )(scb, xs_g, xd_g, xpad, *ws)


def _gnn_body(h_ref, a0_ref, a1_ref, w1, b1, w2, b2, o_ref):
    z = h_ref[...] + a0_ref[...] + a1_ref[...]
    o_ref[...] = (jnp.maximum(z @ w1[...] + b1[...], 0.0)
                  @ w2[...] + b2[...])


def _gnn_dense(h, a0, a1, w1, b1, w2, b2):
    blk = 1024
    return pl.pallas_call(
        _gnn_body,
        grid=(B * N // blk,),
        in_specs=[pl.BlockSpec((blk, NH), lambda i: (i, 0))] * 3
        + [_full(w1.shape), _full(b1.shape), _full(w2.shape), _full(b2.shape)],
        out_specs=pl.BlockSpec((blk, NH), lambda i: (i, 0)),
        out_shape=jax.ShapeDtypeStruct((B * N, NH), jnp.float32),
    )(h, a0, a1, w1, b1, w2, b2)


def _gnn_last_body(h_ref, a0_ref, a1_ref, w1, b1, w2, b2, ow, ob, o_ref):
    z = h_ref[...] + a0_ref[...] + a1_ref[...]
    t = (jnp.maximum(z @ w1[...] + b1[...], 0.0) @ w2[...] + b2[...])
    m = jnp.mean(t, axis=0, keepdims=True)
    o_ref[0] = m @ ow[...] + ob[...]


def _gnn_last(h, a0, a1, w1, b1, w2, b2, ow, ob):
    o3 = pl.pallas_call(
        _gnn_last_body,
        grid=(B,),
        in_specs=[pl.BlockSpec((N, NH), lambda i: (i, 0))] * 3
        + [_full(w.shape) for w in (w1, b1, w2, b2, ow, ob)],
        out_specs=pl.BlockSpec((1, 1, NH), lambda b: (b, 0, 0)),
        out_shape=jax.ShapeDtypeStruct((B, 1, NH), jnp.float32),
    )(h, a0, a1, w1, b1, w2, b2, ow, ob)
    return o3.reshape(B, NH)


_NC, _NS = 2, 16
_NW = _NC * _NS
_ET = B * E
_EW = _ET // _NW
_CH = 128
_NCH = _EW // _CH
_GC = 64
_GNCH = _EW // _GC


def _sc_mesh():
    return plsc.VectorSubcoreMesh(core_axis_name="c", subcore_axis_name="s",
                                  num_cores=_NC, num_subcores=_NS)


def _zero_vmem(buf, nrows):
    zz = jnp.zeros((16,), jnp.float32)

    def row(r, _):
        for c in range(NH // 16):
            buf[r, pl.ds(c * 16, 16)] = zz
        return 0

    lax.fori_loop(0, nrows, row, 0)


def _xgather_body(x2_hbm, ei_hbm, xs_hbm, xd_hbm, srcf_hbm, dstf_hbm,
                  idx_s, idx_d, rows_s, rows_d, sem_s, sem_d):
    cid = lax.axis_index("c")
    sid = lax.axis_index("s")
    wid = sid * _NC + cid
    base = wid * _EW
    b = wid // (_NW // B)
    r0 = base - b * E
    pltpu.sync_copy(ei_hbm.at[pl.ds(b * 2 * E + r0, _EW)], idx_s)
    pltpu.sync_copy(ei_hbm.at[pl.ds(b * 2 * E + E + r0, _EW)], idx_d)
    off = jnp.full((16,), b * N, jnp.int32)

    def addoff(i, _):
        sl = pl.ds(i * 16, 16)
        idx_s[sl] = idx_s[sl] + off
        idx_d[sl] = idx_d[sl] + off
        return 0

    lax.fori_loop(0, _EW // 16, addoff, 0)
    cps = []
    for j in range(_NCH):
        sl = pl.ds(j * _CH, _CH)
        cps.append(pltpu.async_copy(x2_hbm.at[idx_s.at[sl]],
                                    rows_s.at[sl], sem_s))
        cps.append(pltpu.async_copy(x2_hbm.at[idx_d.at[sl]],
                                    rows_d.at[sl], sem_d))
    pltpu.sync_copy(idx_s, srcf_hbm.at[pl.ds(base, _EW)])
    pltpu.sync_copy(idx_d, dstf_hbm.at[pl.ds(base, _EW)])
    for cp in cps:
        cp.wait()
    pltpu.sync_copy(rows_s, xs_hbm.at[pl.ds(base, _EW)])
    pltpu.sync_copy(rows_d, xd_hbm.at[pl.ds(base, _EW)])


def _sc_xgather(x2, ei_flat):
    f = pl.kernel(
        _xgather_body,
        out_type=[jax.ShapeDtypeStruct((_ET, 2), jnp.float32),
                  jax.ShapeDtypeStruct((_ET, 2), jnp.float32),
                  jax.ShapeDtypeStruct((_ET,), jnp.int32),
                  jax.ShapeDtypeStruct((_ET,), jnp.int32)],
        mesh=_sc_mesh(),
        scratch_types=[pltpu.VMEM((_EW,), jnp.int32),
                       pltpu.VMEM((_EW,), jnp.int32),
                       pltpu.VMEM((_EW, 2), jnp.float32),
                       pltpu.VMEM((_EW, 2), jnp.float32),
                       pltpu.SemaphoreType.DMA,
                       pltpu.SemaphoreType.DMA],
        compiler_params=pltpu.CompilerParams(use_tc_tiling_on_sc=False),
    )
    return f(x2, ei_flat)


def _gnnmsg_body(h_hbm, ea_hbm, src_hbm, dst_hbm, out0_hbm, out1_hbm,
                 idx_s, idx_d, rows, eab, zbuf, shared,
                 gsem, esem, ssem):
    cid = lax.axis_index("c")
    sid = lax.axis_index("s")
    wid = sid * _NC + cid
    base = wid * _EW
    stripe = B * N // _NS

    for j in range(_GNCH):
        pltpu.sync_copy(src_hbm.at[pl.ds(base + j * _GC, _GC)], idx_s.at[j])
        pltpu.sync_copy(dst_hbm.at[pl.ds(base + j * _GC, _GC)], idx_d.at[j])
    gcp = [None] * _GNCH
    ecp = [None] * _GNCH
    scp = [None] * _GNCH

    def fire_g(j):
        gcp[j] = pltpu.async_copy(h_hbm.at[idx_s.at[j]],
                                  rows.at[j % 3], gsem[j % 3])

    def fire_e(j):
        ecp[j] = pltpu.async_copy(ea_hbm.at[pl.ds(base + j * _GC, _GC)],
                                  eab.at[j % 2], esem[j % 2])

    fire_g(0), fire_g(1), fire_g(2)
    fire_e(0), fire_e(1)
    _zero_vmem(zbuf, 64)
    for j in range(stripe // 64):
        pltpu.sync_copy(zbuf, shared.at[pl.ds(sid * stripe + j * 64, 64)])
    plsc.subcore_barrier()

    for j in range(_GNCH):
        gcp[j].wait()
        ecp[j].wait()
        rp = rows.at[j % 3]
        ep = eab.at[j % 2]

        def row(r, _):
            for c in range(NH // 16):
                sl = pl.ds(c * 16, 16)
                rp[r, sl] = jnp.maximum(rp[r, sl] + ep[r, sl], 0.0)
            return 0

        lax.fori_loop(0, _GC, row, 0)
        scp[j] = pltpu.async_copy(rp, shared.at[idx_d.at[j]],
                                  ssem[j % 3], add=True)
        if j + 2 < _GNCH:
            fire_e(j + 2)
        if j >= 1 and j + 2 < _GNCH:
            scp[j - 1].wait()
            fire_g(j + 2)
    for j in range(max(0, _GNCH - 3), _GNCH):
        scp[j].wait()
    plsc.subcore_barrier()

    src_sl = shared.at[pl.ds(sid * stripe, stripe)]
    out_sl = pl.ds(sid * stripe, stripe)

    @pl.when(cid == 0)
    def _():
        pltpu.sync_copy(src_sl, out0_hbm.at[out_sl])

    @pl.when(cid == 1)
    def _():
        pltpu.sync_copy(src_sl, out1_hbm.at[out_sl])


def _scatter_body(msg_hbm, dst_hbm, out0_hbm, out1_hbm,
                  idx_d, rows, zbuf, shared, msem, ssem):
    cid = lax.axis_index("c")
    sid = lax.axis_index("s")
    wid = sid * _NC + cid
    base = wid * _EW
    stripe = B * N // _NS

    for j in range(_GNCH):
        pltpu.sync_copy(dst_hbm.at[pl.ds(base + j * _GC, _GC)], idx_d.at[j])
    mcp = [None] * _GNCH
    scp = [None] * _GNCH

    def fire_m(j):
        mcp[j] = pltpu.async_copy(msg_hbm.at[pl.ds(base + j * _GC, _GC)],
                                  rows.at[j % 3], msem[j % 3])

    fire_m(0), fire_m(1), fire_m(2)
    _zero_vmem(zbuf, 64)
    for j in range(stripe // 64):
        pltpu.sync_copy(zbuf, shared.at[pl.ds(sid * stripe + j * 64, 64)])
    plsc.subcore_barrier()

    for j in range(_GNCH):
        mcp[j].wait()
        scp[j] = pltpu.async_copy(rows.at[j % 3], shared.at[idx_d.at[j]],
                                  ssem[j % 3], add=True)
        if j >= 1 and j + 2 < _GNCH:
            scp[j - 1].wait()
            fire_m(j + 2)
    for j in range(max(0, _GNCH - 3), _GNCH):
        scp[j].wait()
    plsc.subcore_barrier()

    src_sl = shared.at[pl.ds(sid * stripe, stripe)]
    out_sl = pl.ds(sid * stripe, stripe)

    @pl.when(cid == 0)
    def _():
        pltpu.sync_copy(src_sl, out0_hbm.at[out_sl])

    @pl.when(cid == 1)
    def _():
        pltpu.sync_copy(src_sl, out1_hbm.at[out_sl])


def _sc_scatter(msg, dst_f):
    f = pl.kernel(
        _scatter_body,
        out_type=[jax.ShapeDtypeStruct((B * N, NH), jnp.float32),
                  jax.ShapeDtypeStruct((B * N, NH), jnp.float32)],
        mesh=_sc_mesh(),
        scratch_types=[pltpu.VMEM((_GNCH, _GC), jnp.int32),
                       pltpu.VMEM((3, _GC, NH), jnp.float32),
                       pltpu.VMEM((64, NH), jnp.float32),
                       pltpu.VMEM_SHARED((B * N, NH), jnp.float32),
                       [pltpu.SemaphoreType.DMA] * 3,
                       [pltpu.SemaphoreType.DMA] * 3],
    )
    return f(msg, dst_f)


def _sc_gnn_msg(h, ea, src_f, dst_f):
    f = pl.kernel(
        _gnnmsg_body,
        out_type=[jax.ShapeDtypeStruct((B * N, NH), jnp.float32),
                  jax.ShapeDtypeStruct((B * N, NH), jnp.float32)],
        mesh=_sc_mesh(),
        scratch_types=[pltpu.VMEM((_GNCH, _GC), jnp.int32),
                       pltpu.VMEM((_GNCH, _GC), jnp.int32),
                       pltpu.VMEM((3, _GC, NH), jnp.float32),
                       pltpu.VMEM((2, _GC, NH), jnp.float32),
                       pltpu.VMEM((64, NH), jnp.float32),
                       pltpu.VMEM_SHARED((B * N, NH), jnp.float32),
                       [pltpu.SemaphoreType.DMA] * 3,
                       [pltpu.SemaphoreType.DMA] * 2,
                       [pltpu.SemaphoreType.DMA] * 3],
    )
    return f(h, ea, src_f, dst_f)


def kernel(x, edge_index, scb, enc1_W1, enc1_b1, enc1_W2, enc1_b2,
           enc2_W1, enc2_b1, enc2_W2, enc2_b2, enc4_W1, enc4_b1,
           enc4_W2, enc4_b2, node_W, node_b, gnn_W1, gnn_b1, gnn_W2,
           gnn_b2, out_W, out_b):
    xs_g, xd_g, src_f, dst_f = _sc_xgather(x.reshape(B * N, 2),
                                           edge_index.reshape(-1))

    b2d = lambda v: v.reshape(1, -1)
    edge_attr, h, msg1 = _edge_encoder(
        scb, xs_g.reshape(B, E, 2), xd_g.reshape(B, E, 2), x,
        enc1_W1, b2d(enc1_b1), enc1_W2, b2d(enc1_b2),
        enc2_W1, enc2_b1.reshape(-1, 1),
        enc2_W2, b2d(enc2_b2), enc4_W1, b2d(enc4_b1), enc4_W2, b2d(enc4_b2),
        node_W, b2d(node_b),
    )
    edge_attr = edge_attr.reshape(B * E, NH)
    h = h.reshape(B * N, NH)
    msg1 = msg1.reshape(B * E, NH)

    a0, a1 = _sc_scatter(msg1, dst_f)
    h = _gnn_dense(h, a0, a1, gnn_W1[0], b2d(gnn_b1[0]),
                   gnn_W2[0], b2d(gnn_b2[0]))
    a0, a1 = _sc_gnn_msg(h, edge_attr, src_f, dst_f)
    h = _gnn_dense(h, a0, a1, gnn_W1[1], b2d(gnn_b1[1]),
                   gnn_W2[1], b2d(gnn_b2[1]))
    a0, a1 = _sc_gnn_msg(h, edge_attr, src_f, dst_f)
    return _gnn_last(h, a0, a1, gnn_W1[2], b2d(gnn_b1[2]),
                     gnn_W2[2], b2d(gnn_b2[2]), out_W, b2d(out_b))


# speedup vs baseline: 1.0163x; 1.0163x over previous
import functools

import jax
import jax.numpy as jnp
from jax import lax
from jax.experimental import pallas as pl
from jax.experimental.pallas import tpu as pltpu
from jax.experimental.pallas import tpu_sc as plsc

B, N, E, BETA = 8, 1024, 2048, 16
NH = 128


def _enc_body(scb_ref, xs_ref, xd_ref, xp_ref,
              e1w1, e1b1, e1w2, e1b2, w1, e2b1c,
              e2w2, e2b2, e4w1, e4b1, e4w2, e4b2, nw, nb,
              ea_ref, h0_ref, m1_ref):
    SCB = jnp.abs(scb_ref[0])
    xs = xs_ref[0][:, 0:2]
    xd = xd_ref[0][:, 0:2]
    dg = lambda a, b, dn: lax.dot_general(a, b, (dn, ((), ())),
                                          preferred_element_type=jnp.float32)
    sA = SCB @ xs
    sB = SCB @ xd
    h1 = (jnp.maximum(sA @ e1w1[0:2, :] + sB @ e1w1[2:4, :] + e1b1[...], 0.0)
          @ e1w2[...] + e1b2[...])
    baseT = dg(w1[0:64, :], h1, ((0,), (1,))) + e2b1c[...]
    bb = [jnp.broadcast_to(baseT[:, k:k + 1], (NH, NH)) for k in range(BETA)]
    epT = (dg(w1[64:66, :], xs, ((0,), (1,)))
           + dg(w1[66:68, :], xd, ((0,), (1,))))
    g1 = xs @ nw[...] + nb[...]
    Ec = 128
    for c in range(E // Ec):
        sl = slice(c * Ec, (c + 1) * Ec)
        epc = epT[:, sl]
        acc = jnp.maximum(SCB[0:1, sl] * epc + bb[0], 0.0)
        for k in range(1, BETA):
            acc = acc + jnp.maximum(SCB[k:k + 1, sl] * epc + bb[k], 0.0)
        emb = dg(acc, e2w2[...], ((0,), (0,))) + BETA * e2b2[...]
        ea = (jnp.maximum(emb @ e4w1[...] + e4b1[...], 0.0)
              @ e4w2[...] + e4b2[...])
        ea_ref[0, sl, :] = ea
        m1_ref[0, sl, :] = jnp.maximum(g1[sl, :] + ea, 0.0)
    h0_ref[0] = xp_ref[0] @ nw[...] + nb[...]


def _full(shape):
    nd = len(shape)
    return pl.BlockSpec(shape, lambda *_, _n=nd: (0,) * _n)


def _edge_encoder(scb, xs_g, xd_g, xpad, *ws):
    return pl.pallas_call(
        _enc_body,
        grid=(B,),
        in_specs=[
            pl.BlockSpec((1, BETA, E), lambda b: (b, 0, 0)),
            pl.BlockSpec((1, E, 16), lambda b: (b, 0, 0)),
            pl.BlockSpec((1, E, 16), lambda b: (b, 0, 0)),
            pl.BlockSpec((1, N, 2), lambda b: (b, 0, 0)),
        ] + [_full(w.shape) for w in ws],
        out_specs=[pl.BlockSpec((1, E, NH), lambda b: (b, 0, 0)),
                   pl.BlockSpec((1, N, NH), lambda b: (b, 0, 0)),
                   pl.BlockSpec((1, E, NH), lambda b: (b, 0, 0))],
        out_shape=[jax.ShapeDtypeStruct((B, E, NH), jnp.float32),
                   jax.ShapeDtypeStruct((B, N, NH), jnp.float32),
                   jax.ShapeDtypeStruct((B, E, NH), jnp.float32)],
    )(scb, xs_g, xd_g, xpad, *ws)


def _gnn_body(h_ref, a0_ref, a1_ref, w1, b1, w2, b2, o_ref):
    z = h_ref[...] + a0_ref[...] + a1_ref[...]
    o_ref[...] = (jnp.maximum(z @ w1[...] + b1[...], 0.0)
                  @ w2[...] + b2[...])


def _gnn_dense(h, a0, a1, w1, b1, w2, b2):
    blk = 1024
    return pl.pallas_call(
        _gnn_body,
        grid=(B * N // blk,),
        in_specs=[pl.BlockSpec((blk, NH), lambda i: (i, 0))] * 3
        + [_full(w1.shape), _full(b1.shape), _full(w2.shape), _full(b2.shape)],
        out_specs=pl.BlockSpec((blk, NH), lambda i: (i, 0)),
        out_shape=jax.ShapeDtypeStruct((B * N, NH), jnp.float32),
    )(h, a0, a1, w1, b1, w2, b2)


def _gnn_last_body(h_ref, a0_ref, a1_ref, w1, b1, w2, b2, ow, ob, o_ref):
    z = h_ref[...] + a0_ref[...] + a1_ref[...]
    t = (jnp.maximum(z @ w1[...] + b1[...], 0.0) @ w2[...] + b2[...])
    m = jnp.mean(t, axis=0, keepdims=True)
    o_ref[0] = m @ ow[...] + ob[...]


def _gnn_last(h, a0, a1, w1, b1, w2, b2, ow, ob):
    o3 = pl.pallas_call(
        _gnn_last_body,
        grid=(B,),
        in_specs=[pl.BlockSpec((N, NH), lambda i: (i, 0))] * 3
        + [_full(w.shape) for w in (w1, b1, w2, b2, ow, ob)],
        out_specs=pl.BlockSpec((1, 1, NH), lambda b: (b, 0, 0)),
        out_shape=jax.ShapeDtypeStruct((B, 1, NH), jnp.float32),
    )(h, a0, a1, w1, b1, w2, b2, ow, ob)
    return o3.reshape(B, NH)


_NC, _NS = 2, 16
_NW = _NC * _NS
_ET = B * E
_EW = _ET // _NW
_CH = 128
_NCH = _EW // _CH
_GC = 64
_GNCH = _EW // _GC


def _sc_mesh():
    return plsc.VectorSubcoreMesh(core_axis_name="c", subcore_axis_name="s",
                                  num_cores=_NC, num_subcores=_NS)


def _zero_vmem(buf, nrows):
    zz = jnp.zeros((16,), jnp.float32)

    def row(r, _):
        for c in range(NH // 16):
            buf[r, pl.ds(c * 16, 16)] = zz
        return 0

    lax.fori_loop(0, nrows, row, 0)


def _xgather_body(xpad_hbm, ei_hbm, xs_hbm, xd_hbm, srcf_hbm, dstf_hbm,
                  idx_s, idx_d, rows_s, rows_d, sem_s, sem_d):
    cid = lax.axis_index("c")
    sid = lax.axis_index("s")
    wid = sid * _NC + cid
    base = wid * _EW
    b = wid // (_NW // B)
    r0 = base - b * E
    pltpu.sync_copy(ei_hbm.at[pl.ds(b * 2 * E + r0, _EW)], idx_s)
    pltpu.sync_copy(ei_hbm.at[pl.ds(b * 2 * E + E + r0, _EW)], idx_d)
    off = jnp.full((16,), b * N, jnp.int32)

    def addoff(i, _):
        sl = pl.ds(i * 16, 16)
        idx_s[sl] = idx_s[sl] + off
        idx_d[sl] = idx_d[sl] + off
        return 0

    lax.fori_loop(0, _EW // 16, addoff, 0)
    cps = []
    for j in range(_NCH):
        sl = pl.ds(j * _CH, _CH)
        cps.append(pltpu.async_copy(xpad_hbm.at[idx_s.at[sl]],
                                    rows_s.at[sl], sem_s))
        cps.append(pltpu.async_copy(xpad_hbm.at[idx_d.at[sl]],
                                    rows_d.at[sl], sem_d))
    pltpu.sync_copy(idx_s, srcf_hbm.at[pl.ds(base, _EW)])
    pltpu.sync_copy(idx_d, dstf_hbm.at[pl.ds(base, _EW)])
    for cp in cps:
        cp.wait()
    pltpu.sync_copy(rows_s, xs_hbm.at[pl.ds(base, _EW)])
    pltpu.sync_copy(rows_d, xd_hbm.at[pl.ds(base, _EW)])


def _sc_xgather(xpad, ei_flat):
    f = pl.kernel(
        _xgather_body,
        out_type=[jax.ShapeDtypeStruct((_ET, 16), jnp.float32),
                  jax.ShapeDtypeStruct((_ET, 16), jnp.float32),
                  jax.ShapeDtypeStruct((_ET,), jnp.int32),
                  jax.ShapeDtypeStruct((_ET,), jnp.int32)],
        mesh=_sc_mesh(),
        scratch_types=[pltpu.VMEM((_EW,), jnp.int32),
                       pltpu.VMEM((_EW,), jnp.int32),
                       pltpu.VMEM((_EW, 16), jnp.float32),
                       pltpu.VMEM((_EW, 16), jnp.float32),
                       pltpu.SemaphoreType.DMA,
                       pltpu.SemaphoreType.DMA],
        compiler_params=pltpu.CompilerParams(use_tc_tiling_on_sc=False),
    )
    return f(xpad, ei_flat)


def _gnnmsg_body(h_hbm, ea_hbm, src_hbm, dst_hbm, out0_hbm, out1_hbm,
                 idx_s, idx_d, rows, eab, zbuf, shared,
                 gsem, esem, ssem):
    cid = lax.axis_index("c")
    sid = lax.axis_index("s")
    wid = sid * _NC + cid
    base = wid * _EW
    stripe = B * N // _NS

    for j in range(_GNCH):
        pltpu.sync_copy(src_hbm.at[pl.ds(base + j * _GC, _GC)], idx_s.at[j])
        pltpu.sync_copy(dst_hbm.at[pl.ds(base + j * _GC, _GC)], idx_d.at[j])
    gcp = [None] * _GNCH
    ecp = [None] * _GNCH
    scp = [None] * _GNCH

    def fire_g(j):
        gcp[j] = pltpu.async_copy(h_hbm.at[idx_s.at[j]],
                                  rows.at[j % 3], gsem[j % 3])

    def fire_e(j):
        ecp[j] = pltpu.async_copy(ea_hbm.at[pl.ds(base + j * _GC, _GC)],
                                  eab.at[j % 2], esem[j % 2])

    fire_g(0), fire_g(1), fire_g(2)
    fire_e(0), fire_e(1)
    _zero_vmem(zbuf, 64)
    for j in range(stripe // 64):
        pltpu.sync_copy(zbuf, shared.at[pl.ds(sid * stripe + j * 64, 64)])
    plsc.subcore_barrier()

    for j in range(_GNCH):
        gcp[j].wait()
        ecp[j].wait()
        rp = rows.at[j % 3]
        ep = eab.at[j % 2]

        def row(r, _):
            for c in range(NH // 16):
                sl = pl.ds(c * 16, 16)
                rp[r, sl] = jnp.maximum(rp[r, sl] + ep[r, sl], 0.0)
            return 0

        lax.fori_loop(0, _GC, row, 0)
        scp[j] = pltpu.async_copy(rp, shared.at[idx_d.at[j]],
                                  ssem[j % 3], add=True)
        if j + 2 < _GNCH:
            fire_e(j + 2)
        if j >= 1 and j + 2 < _GNCH:
            scp[j - 1].wait()
            fire_g(j + 2)
    for j in range(max(0, _GNCH - 3), _GNCH):
        scp[j].wait()
    plsc.subcore_barrier()

    src_sl = shared.at[pl.ds(sid * stripe, stripe)]
    out_sl = pl.ds(sid * stripe, stripe)

    @pl.when(cid == 0)
    def _():
        pltpu.sync_copy(src_sl, out0_hbm.at[out_sl])

    @pl.when(cid == 1)
    def _():
        pltpu.sync_copy(src_sl, out1_hbm.at[out_sl])


def _scatter_body(msg_hbm, dst_hbm, out0_hbm, out1_hbm,
                  idx_d, rows, zbuf, shared, msem, ssem):
    cid = lax.axis_index("c")
    sid = lax.axis_index("s")
    wid = sid * _NC + cid
    base = wid * _EW
    stripe = B * N // _NS

    for j in range(_GNCH):
        pltpu.sync_copy(dst_hbm.at[pl.ds(base + j * _GC, _GC)], idx_d.at[j])
    mcp = [None] * _GNCH
    scp = [None] * _GNCH

    def fire_m(j):
        mcp[j] = pltpu.async_copy(msg_hbm.at[pl.ds(base + j * _GC, _GC)],
                                  rows.at[j % 3], msem[j % 3])

    fire_m(0), fire_m(1), fire_m(2)
    _zero_vmem(zbuf, 64)
    for j in range(stripe // 64):
        pltpu.sync_copy(zbuf, shared.at[pl.ds(sid * stripe + j * 64, 64)])
    plsc.subcore_barrier()

    for j in range(_GNCH):
        mcp[j].wait()
        scp[j] = pltpu.async_copy(rows.at[j % 3], shared.at[idx_d.at[j]],
                                  ssem[j % 3], add=True)
        if j >= 1 and j + 2 < _GNCH:
            scp[j - 1].wait()
            fire_m(j + 2)
    for j in range(max(0, _GNCH - 3), _GNCH):
        scp[j].wait()
    plsc.subcore_barrier()

    src_sl = shared.at[pl.ds(sid * stripe, stripe)]
    out_sl = pl.ds(sid * stripe, stripe)

    @pl.when(cid == 0)
    def _():
        pltpu.sync_copy(src_sl, out0_hbm.at[out_sl])

    @pl.when(cid == 1)
    def _():
        pltpu.sync_copy(src_sl, out1_hbm.at[out_sl])


def _sc_scatter(msg, dst_f):
    f = pl.kernel(
        _scatter_body,
        out_type=[jax.ShapeDtypeStruct((B * N, NH), jnp.float32),
                  jax.ShapeDtypeStruct((B * N, NH), jnp.float32)],
        mesh=_sc_mesh(),
        scratch_types=[pltpu.VMEM((_GNCH, _GC), jnp.int32),
                       pltpu.VMEM((3, _GC, NH), jnp.float32),
                       pltpu.VMEM((64, NH), jnp.float32),
                       pltpu.VMEM_SHARED((B * N, NH), jnp.float32),
                       [pltpu.SemaphoreType.DMA] * 3,
                       [pltpu.SemaphoreType.DMA] * 3],
    )
    return f(msg, dst_f)


def _sc_gnn_msg(h, ea, src_f, dst_f):
    f = pl.kernel(
        _gnnmsg_body,
        out_type=[jax.ShapeDtypeStruct((B * N, NH), jnp.float32),
                  jax.ShapeDtypeStruct((B * N, NH), jnp.float32)],
        mesh=_sc_mesh(),
        scratch_types=[pltpu.VMEM((_GNCH, _GC), jnp.int32),
                       pltpu.VMEM((_GNCH, _GC), jnp.int32),
                       pltpu.VMEM((3, _GC, NH), jnp.float32),
                       pltpu.VMEM((2, _GC, NH), jnp.float32),
                       pltpu.VMEM((64, NH), jnp.float32),
                       pltpu.VMEM_SHARED((B * N, NH), jnp.float32),
                       [pltpu.SemaphoreType.DMA] * 3,
                       [pltpu.SemaphoreType.DMA] * 2,
                       [pltpu.SemaphoreType.DMA] * 3],
    )
    return f(h, ea, src_f, dst_f)


def kernel(x, edge_index, scb, enc1_W1, enc1_b1, enc1_W2, enc1_b2,
           enc2_W1, enc2_b1, enc2_W2, enc2_b2, enc4_W1, enc4_b1,
           enc4_W2, enc4_b2, node_W, node_b, gnn_W1, gnn_b1, gnn_W2,
           gnn_b2, out_W, out_b):
    xpad = jnp.pad(x.reshape(B * N, 2), ((0, 0), (0, 14)))
    xs_g, xd_g, src_f, dst_f = _sc_xgather(xpad, edge_index.reshape(-1))

    b2d = lambda v: v.reshape(1, -1)
    edge_attr, h, msg1 = _edge_encoder(
        scb, xs_g.reshape(B, E, 16), xd_g.reshape(B, E, 16), x,
        enc1_W1, b2d(enc1_b1), enc1_W2, b2d(enc1_b2),
        enc2_W1, enc2_b1.reshape(-1, 1),
        enc2_W2, b2d(enc2_b2), enc4_W1, b2d(enc4_b1), enc4_W2, b2d(enc4_b2),
        node_W, b2d(node_b),
    )
    edge_attr = edge_attr.reshape(B * E, NH)
    h = h.reshape(B * N, NH)
    msg1 = msg1.reshape(B * E, NH)

    a0, a1 = _sc_scatter(msg1, dst_f)
    h = _gnn_dense(h, a0, a1, gnn_W1[0], b2d(gnn_b1[0]),
                   gnn_W2[0], b2d(gnn_b2[0]))
    a0, a1 = _sc_gnn_msg(h, edge_attr, src_f, dst_f)
    h = _gnn_dense(h, a0, a1, gnn_W1[1], b2d(gnn_b1[1]),
                   gnn_W2[1], b2d(gnn_b2[1]))
    a0, a1 = _sc_gnn_msg(h, edge_attr, src_f, dst_f)
    return _gnn_last(h, a0, a1, gnn_W1[2], b2d(gnn_b1[2]),
                     gnn_W2[2], b2d(gnn_b2[2]), out_W, b2d(out_b))


# speedup vs baseline: 1.0953x; 1.0778x over previous
import functools

import jax
import jax.numpy as jnp
from jax import lax
from jax.experimental import pallas as pl
from jax.experimental.pallas import tpu as pltpu
from jax.experimental.pallas import tpu_sc as plsc

B, N, E, BETA = 8, 1024, 2048, 16
NH = 128


def _enc_body(scb_ref, xs_ref, xd_ref, xp_ref,
              e1w1, e1b1, e1w2, e1b2, w1, e2b1c,
              e2w2, e2b2, e4w1, e4b1, e4w2, e4b2, nw, nb,
              ea_ref, h0_ref, m1_ref):
    SCB = jnp.abs(scb_ref[0])
    xs = xs_ref[0][:, 0:2]
    xd = xd_ref[0][:, 0:2]
    dg = lambda a, b, dn: lax.dot_general(a, b, (dn, ((), ())),
                                          preferred_element_type=jnp.float32)
    sA = SCB @ xs
    sB = SCB @ xd
    h1 = (jnp.maximum(sA @ e1w1[0:2, :] + sB @ e1w1[2:4, :] + e1b1[...], 0.0)
          @ e1w2[...] + e1b2[...])
    baseT = dg(w1[0:64, :], h1, ((0,), (1,))) + e2b1c[...]
    bb = [jnp.broadcast_to(baseT[:, k:k + 1], (NH, NH)) for k in range(BETA)]
    epT = (dg(w1[64:66, :], xs, ((0,), (1,)))
           + dg(w1[66:68, :], xd, ((0,), (1,))))
    g1 = xs @ nw[...] + nb[...]
    Ec = 128
    for c in range(E // Ec):
        sl = slice(c * Ec, (c + 1) * Ec)
        epc = epT[:, sl]
        acc = jnp.maximum(SCB[0:1, sl] * epc + bb[0], 0.0)
        for k in range(1, BETA):
            acc = acc + jnp.maximum(SCB[k:k + 1, sl] * epc + bb[k], 0.0)
        emb = dg(acc, e2w2[...], ((0,), (0,))) + BETA * e2b2[...]
        ea = (jnp.maximum(emb @ e4w1[...] + e4b1[...], 0.0)
              @ e4w2[...] + e4b2[...])
        ea_ref[0, sl, :] = ea
        m1_ref[0, sl, :] = jnp.maximum(g1[sl, :] + ea, 0.0)
    h0_ref[0] = xp_ref[0] @ nw[...] + nb[...]


def _full(shape):
    nd = len(shape)
    return pl.BlockSpec(shape, lambda *_, _n=nd: (0,) * _n)


def _edge_encoder(scb, xs_g, xd_g, xpad, *ws):
    return pl.pallas_call(
        _enc_body,
        grid=(B,),
        in_specs=[
            pl.BlockSpec((1, BETA, E), lambda b: (b, 0, 0)),
            pl.BlockSpec((1, E, 16), lambda b: (b, 0, 0)),
            pl.BlockSpec((1, E, 16), lambda b: (b, 0, 0)),
            pl.BlockSpec((1, N, 2), lambda b: (b, 0, 0)),
        ] + [_full(w.shape) for w in ws],
        out_specs=[pl.BlockSpec((1, E, NH), lambda b: (b, 0, 0)),
                   pl.BlockSpec((1, N, NH), lambda b: (b, 0, 0)),
                   pl.BlockSpec((1, E, NH), lambda b: (b, 0, 0))],
        out_shape=[jax.ShapeDtypeStruct((B, E, NH), jnp.float32),
                   jax.ShapeDtypeStruct((B, N, NH), jnp.float32),
                   jax.ShapeDtypeStruct((B, E, NH), jnp.float32)],
    )(scb, xs_g, xd_g, xpad, *ws)


def _gnn_body(h_ref, a0_ref, a1_ref, w1, b1, w2, b2, o_ref):
    z = h_ref[...] + a0_ref[...] + a1_ref[...]
    o_ref[...] = (jnp.maximum(z @ w1[...] + b1[...], 0.0)
                  @ w2[...] + b2[...])


def _gnn_dense(h, a0, a1, w1, b1, w2, b2):
    blk = 1024
    return pl.pallas_call(
        _gnn_body,
        grid=(B * N // blk,),
        in_specs=[pl.BlockSpec((blk, NH), lambda i: (i, 0))] * 3
        + [_full(w1.shape), _full(b1.shape), _full(w2.shape), _full(b2.shape)],
        out_specs=pl.BlockSpec((blk, NH), lambda i: (i, 0)),
        out_shape=jax.ShapeDtypeStruct((B * N, NH), jnp.float32),
    )(h, a0, a1, w1, b1, w2, b2)


def _gnn_last_body(h_ref, a0_ref, a1_ref, w1, b1, w2, b2, ow, ob, o_ref):
    z = h_ref[...] + a0_ref[...] + a1_ref[...]
    t = (jnp.maximum(z @ w1[...] + b1[...], 0.0) @ w2[...] + b2[...])
    m = jnp.mean(t, axis=0, keepdims=True)
    o_ref[0] = m @ ow[...] + ob[...]


def _gnn_last(h, a0, a1, w1, b1, w2, b2, ow, ob):
    o3 = pl.pallas_call(
        _gnn_last_body,
        grid=(B,),
        in_specs=[pl.BlockSpec((N, NH), lambda i: (i, 0))] * 3
        + [_full(w.shape) for w in (w1, b1, w2, b2, ow, ob)],
        out_specs=pl.BlockSpec((1, 1, NH), lambda b: (b, 0, 0)),
        out_shape=jax.ShapeDtypeStruct((B, 1, NH), jnp.float32),
    )(h, a0, a1, w1, b1, w2, b2, ow, ob)
    return o3.reshape(B, NH)


_NC, _NS = 2, 16
_NW = _NC * _NS
_ET = B * E
_EW = _ET // _NW
_CH = 128
_NCH = _EW // _CH
_GC = 64
_GNCH = _EW // _GC


def _sc_mesh():
    return plsc.VectorSubcoreMesh(core_axis_name="c", subcore_axis_name="s",
                                  num_cores=_NC, num_subcores=_NS)


def _zero_vmem(buf, nrows):
    zz = jnp.zeros((16,), jnp.float32)

    def row(r, _):
        for c in range(NH // 16):
            buf[r, pl.ds(c * 16, 16)] = zz
        return 0

    lax.fori_loop(0, nrows, row, 0)


def _xgather_body(xpad_hbm, ei_hbm, xs_hbm, xd_hbm, srcf_hbm, dstf_hbm,
                  idx_s, idx_d, rows_s, rows_d, sem_s, sem_d):
    cid = lax.axis_index("c")
    sid = lax.axis_index("s")
    wid = sid * _NC + cid
    base = wid * _EW
    b = wid // (_NW // B)
    r0 = base - b * E
    nr = _EW // _GC
    pltpu.sync_copy(ei_hbm.at[pl.ds(b * 2 * (E // _GC) + r0 // _GC, nr)],
                    idx_s)
    pltpu.sync_copy(
        ei_hbm.at[pl.ds(b * 2 * (E // _GC) + E // _GC + r0 // _GC, nr)],
        idx_d)
    off = jnp.full((16,), b * N, jnp.int32)

    def addoff(r, _):
        for c in range(_GC // 16):
            sl = pl.ds(c * 16, 16)
            idx_s[r, sl] = idx_s[r, sl] + off
            idx_d[r, sl] = idx_d[r, sl] + off
        return 0

    lax.fori_loop(0, nr, addoff, 0)
    cps = []
    for j in range(nr):
        sl = pl.ds(j * _GC, _GC)
        cps.append(pltpu.async_copy(xpad_hbm.at[idx_s.at[j]],
                                    rows_s.at[sl], sem_s))
        cps.append(pltpu.async_copy(xpad_hbm.at[idx_d.at[j]],
                                    rows_d.at[sl], sem_d))
    pltpu.sync_copy(idx_s, srcf_hbm.at[pl.ds(wid * nr, nr)])
    pltpu.sync_copy(idx_d, dstf_hbm.at[pl.ds(wid * nr, nr)])
    for cp in cps:
        cp.wait()
    pltpu.sync_copy(rows_s, xs_hbm.at[pl.ds(base, _EW)])
    pltpu.sync_copy(rows_d, xd_hbm.at[pl.ds(base, _EW)])


def _sc_xgather(xpad, ei_flat):
    f = pl.kernel(
        _xgather_body,
        out_type=[jax.ShapeDtypeStruct((_ET, 16), jnp.float32),
                  jax.ShapeDtypeStruct((_ET, 16), jnp.float32),
                  jax.ShapeDtypeStruct((_ET // _GC, _GC), jnp.int32),
                  jax.ShapeDtypeStruct((_ET // _GC, _GC), jnp.int32)],
        mesh=_sc_mesh(),
        scratch_types=[pltpu.VMEM((_EW // _GC, _GC), jnp.int32),
                       pltpu.VMEM((_EW // _GC, _GC), jnp.int32),
                       pltpu.VMEM((_EW, 16), jnp.float32),
                       pltpu.VMEM((_EW, 16), jnp.float32),
                       pltpu.SemaphoreType.DMA,
                       pltpu.SemaphoreType.DMA],
        compiler_params=pltpu.CompilerParams(use_tc_tiling_on_sc=False),
    )
    return f(xpad, ei_flat)


def _gnnmsg_body(h_hbm, ea_hbm, src_hbm, dst_hbm, out0_hbm, out1_hbm,
                 idx_s, idx_d, rows, eab, zbuf, shared,
                 gsem, esem, ssem):
    cid = lax.axis_index("c")
    sid = lax.axis_index("s")
    wid = sid * _NC + cid
    base = wid * _EW
    stripe = B * N // _NS

    pltpu.sync_copy(src_hbm.at[pl.ds(wid * _GNCH, _GNCH)], idx_s)
    pltpu.sync_copy(dst_hbm.at[pl.ds(wid * _GNCH, _GNCH)], idx_d)
    gcp = [None] * _GNCH
    ecp = [None] * _GNCH
    scp = [None] * _GNCH

    def fire_g(j):
        gcp[j] = pltpu.async_copy(h_hbm.at[idx_s.at[j]],
                                  rows.at[j % 3], gsem[j % 3])

    def fire_e(j):
        ecp[j] = pltpu.async_copy(ea_hbm.at[pl.ds(base + j * _GC, _GC)],
                                  eab.at[j % 2], esem[j % 2])

    fire_g(0), fire_g(1), fire_g(2)
    fire_e(0), fire_e(1)
    _zero_vmem(zbuf, 64)
    for j in range(stripe // 64):
        pltpu.sync_copy(zbuf, shared.at[pl.ds(sid * stripe + j * 64, 64)])
    plsc.subcore_barrier()

    for j in range(_GNCH):
        gcp[j].wait()
        ecp[j].wait()
        rp = rows.at[j % 3]
        ep = eab.at[j % 2]

        def row(r, _):
            for c in range(NH // 16):
                sl = pl.ds(c * 16, 16)
                rp[r, sl] = jnp.maximum(rp[r, sl] + ep[r, sl], 0.0)
            return 0

        lax.fori_loop(0, _GC, row, 0)
        scp[j] = pltpu.async_copy(rp, shared.at[idx_d.at[j]],
                                  ssem[j % 3], add=True)
        if j + 2 < _GNCH:
            fire_e(j + 2)
        if j >= 1 and j + 2 < _GNCH:
            scp[j - 1].wait()
            fire_g(j + 2)
    for j in range(max(0, _GNCH - 3), _GNCH):
        scp[j].wait()
    plsc.subcore_barrier()

    src_sl = shared.at[pl.ds(sid * stripe, stripe)]
    out_sl = pl.ds(sid * stripe, stripe)

    @pl.when(cid == 0)
    def _():
        pltpu.sync_copy(src_sl, out0_hbm.at[out_sl])

    @pl.when(cid == 1)
    def _():
        pltpu.sync_copy(src_sl, out1_hbm.at[out_sl])


def _scatter_body(msg_hbm, dst_hbm, out0_hbm, out1_hbm,
                  idx_d, rows, zbuf, shared, msem, ssem):
    cid = lax.axis_index("c")
    sid = lax.axis_index("s")
    wid = sid * _NC + cid
    base = wid * _EW
    stripe = B * N // _NS

    pltpu.sync_copy(dst_hbm.at[pl.ds(wid * _GNCH, _GNCH)], idx_d)
    mcp = [None] * _GNCH
    scp = [None] * _GNCH

    def fire_m(j):
        mcp[j] = pltpu.async_copy(msg_hbm.at[pl.ds(base + j * _GC, _GC)],
                                  rows.at[j % 3], msem[j % 3])

    fire_m(0), fire_m(1), fire_m(2)
    _zero_vmem(zbuf, 64)
    for j in range(stripe // 64):
        pltpu.sync_copy(zbuf, shared.at[pl.ds(sid * stripe + j * 64, 64)])
    plsc.subcore_barrier()

    for j in range(_GNCH):
        mcp[j].wait()
        scp[j] = pltpu.async_copy(rows.at[j % 3], shared.at[idx_d.at[j]],
                                  ssem[j % 3], add=True)
        if j >= 1 and j + 2 < _GNCH:
            scp[j - 1].wait()
            fire_m(j + 2)
    for j in range(max(0, _GNCH - 3), _GNCH):
        scp[j].wait()
    plsc.subcore_barrier()

    src_sl = shared.at[pl.ds(sid * stripe, stripe)]
    out_sl = pl.ds(sid * stripe, stripe)

    @pl.when(cid == 0)
    def _():
        pltpu.sync_copy(src_sl, out0_hbm.at[out_sl])

    @pl.when(cid == 1)
    def _():
        pltpu.sync_copy(src_sl, out1_hbm.at[out_sl])


def _sc_scatter(msg, dst_f):
    f = pl.kernel(
        _scatter_body,
        out_type=[jax.ShapeDtypeStruct((B * N, NH), jnp.float32),
                  jax.ShapeDtypeStruct((B * N, NH), jnp.float32)],
        mesh=_sc_mesh(),
        scratch_types=[pltpu.VMEM((_GNCH, _GC), jnp.int32),
                       pltpu.VMEM((3, _GC, NH), jnp.float32),
                       pltpu.VMEM((64, NH), jnp.float32),
                       pltpu.VMEM_SHARED((B * N, NH), jnp.float32),
                       [pltpu.SemaphoreType.DMA] * 3,
                       [pltpu.SemaphoreType.DMA] * 3],
    )
    return f(msg, dst_f)


def _sc_gnn_msg(h, ea, src_f, dst_f):
    f = pl.kernel(
        _gnnmsg_body,
        out_type=[jax.ShapeDtypeStruct((B * N, NH), jnp.float32),
                  jax.ShapeDtypeStruct((B * N, NH), jnp.float32)],
        mesh=_sc_mesh(),
        scratch_types=[pltpu.VMEM((_GNCH, _GC), jnp.int32),
                       pltpu.VMEM((_GNCH, _GC), jnp.int32),
                       pltpu.VMEM((3, _GC, NH), jnp.float32),
                       pltpu.VMEM((2, _GC, NH), jnp.float32),
                       pltpu.VMEM((64, NH), jnp.float32),
                       pltpu.VMEM_SHARED((B * N, NH), jnp.float32),
                       [pltpu.SemaphoreType.DMA] * 3,
                       [pltpu.SemaphoreType.DMA] * 2,
                       [pltpu.SemaphoreType.DMA] * 3],
    )
    return f(h, ea, src_f, dst_f)


def kernel(x, edge_index, scb, enc1_W1, enc1_b1, enc1_W2, enc1_b2,
           enc2_W1, enc2_b1, enc2_W2, enc2_b2, enc4_W1, enc4_b1,
           enc4_W2, enc4_b2, node_W, node_b, gnn_W1, gnn_b1, gnn_W2,
           gnn_b2, out_W, out_b):
    xpad = jnp.pad(x.reshape(B * N, 2), ((0, 0), (0, 14)))
    xs_g, xd_g, src_f, dst_f = _sc_xgather(xpad,
                                            edge_index.reshape(-1, _GC))

    b2d = lambda v: v.reshape(1, -1)
    edge_attr, h, msg1 = _edge_encoder(
        scb, xs_g.reshape(B, E, 16), xd_g.reshape(B, E, 16), x,
        enc1_W1, b2d(enc1_b1), enc1_W2, b2d(enc1_b2),
        enc2_W1, enc2_b1.reshape(-1, 1),
        enc2_W2, b2d(enc2_b2), enc4_W1, b2d(enc4_b1), enc4_W2, b2d(enc4_b2),
        node_W, b2d(node_b),
    )
    edge_attr = edge_attr.reshape(B * E, NH)
    h = h.reshape(B * N, NH)
    msg1 = msg1.reshape(B * E, NH)

    a0, a1 = _sc_scatter(msg1, dst_f)
    h = _gnn_dense(h, a0, a1, gnn_W1[0], b2d(gnn_b1[0]),
                   gnn_W2[0], b2d(gnn_b2[0]))
    a0, a1 = _sc_gnn_msg(h, edge_attr, src_f, dst_f)
    h = _gnn_dense(h, a0, a1, gnn_W1[1], b2d(gnn_b1[1]),
                   gnn_W2[1], b2d(gnn_b2[1]))
    a0, a1 = _sc_gnn_msg(h, edge_attr, src_f, dst_f)
    return _gnn_last(h, a0, a1, gnn_W1[2], b2d(gnn_b1[2]),
                     gnn_W2[2], b2d(gnn_b2[2]), out_W, b2d(out_b))


# speedup vs baseline: 1.1951x; 1.0911x over previous
import functools

import jax
import jax.numpy as jnp
from jax import lax
from jax.experimental import pallas as pl
from jax.experimental.pallas import tpu as pltpu
from jax.experimental.pallas import tpu_sc as plsc

B, N, E, BETA = 8, 1024, 2048, 16
NH = 128


def _enc_body(scb_ref, xs_ref, xd_ref, xp_ref,
              e1w1, e1b1, e1w2, e1b2, w1, e2b1c,
              e2w2, e2b2, e4w1, e4b1, e4w2, e4b2, nw, nb,
              ea_ref, h0_ref, m1_ref):
    SCB = jnp.abs(scb_ref[0])
    xs = xs_ref[0][:, 0:2]
    xd = xd_ref[0][:, 0:2]
    dg = lambda a, b, dn: lax.dot_general(a, b, (dn, ((), ())),
                                          preferred_element_type=jnp.float32)
    sA = SCB @ xs
    sB = SCB @ xd
    h1 = (jnp.maximum(sA @ e1w1[0:2, :] + sB @ e1w1[2:4, :] + e1b1[...], 0.0)
          @ e1w2[...] + e1b2[...])
    baseT = dg(w1[0:64, :], h1, ((0,), (1,))) + e2b1c[...]
    Ec = 256
    bb = [jnp.broadcast_to(baseT[:, k:k + 1], (NH, Ec)) for k in range(BETA)]
    epT = (dg(w1[64:66, :], xs, ((0,), (1,)))
           + dg(w1[66:68, :], xd, ((0,), (1,))))
    g1 = xs @ nw[...] + nb[...]
    for c in range(E // Ec):
        sl = slice(c * Ec, (c + 1) * Ec)
        epc = epT[:, sl]
        acc = jnp.maximum(SCB[0:1, sl] * epc + bb[0], 0.0)
        for k in range(1, BETA):
            acc = acc + jnp.maximum(SCB[k:k + 1, sl] * epc + bb[k], 0.0)
        emb = dg(acc, e2w2[...], ((0,), (0,))) + BETA * e2b2[...]
        ea = (jnp.maximum(emb @ e4w1[...] + e4b1[...], 0.0)
              @ e4w2[...] + e4b2[...])
        ea_ref[0, sl, :] = ea
        m1_ref[0, sl, :] = jnp.maximum(g1[sl, :] + ea, 0.0)
    h0_ref[0] = xp_ref[0] @ nw[...] + nb[...]


def _full(shape):
    nd = len(shape)
    return pl.BlockSpec(shape, lambda *_, _n=nd: (0,) * _n)


def _edge_encoder(scb, xs_g, xd_g, xpad, *ws):
    return pl.pallas_call(
        _enc_body,
        grid=(B,),
        in_specs=[
            pl.BlockSpec((1, BETA, E), lambda b: (b, 0, 0)),
            pl.BlockSpec((1, E, 16), lambda b: (b, 0, 0)),
            pl.BlockSpec((1, E, 16), lambda b: (b, 0, 0)),
            pl.BlockSpec((1, N, 2), lambda b: (b, 0, 0)),
        ] + [_full(w.shape) for w in ws],
        out_specs=[pl.BlockSpec((1, E, NH), lambda b: (b, 0, 0)),
                   pl.BlockSpec((1, N, NH), lambda b: (b, 0, 0)),
                   pl.BlockSpec((1, E, NH), lambda b: (b, 0, 0))],
        out_shape=[jax.ShapeDtypeStruct((B, E, NH), jnp.float32),
                   jax.ShapeDtypeStruct((B, N, NH), jnp.float32),
                   jax.ShapeDtypeStruct((B, E, NH), jnp.float32)],
    )(scb, xs_g, xd_g, xpad, *ws)


def _gnn_body(h_ref, a0_ref, a1_ref, w1, b1, w2, b2, o_ref):
    z = h_ref[...] + a0_ref[...] + a1_ref[...]
    o_ref[...] = (jnp.maximum(z @ w1[...] + b1[...], 0.0)
                  @ w2[...] + b2[...])


def _gnn_dense(h, a0, a1, w1, b1, w2, b2):
    blk = 1024
    return pl.pallas_call(
        _gnn_body,
        grid=(B * N // blk,),
        in_specs=[pl.BlockSpec((blk, NH), lambda i: (i, 0))] * 3
        + [_full(w1.shape), _full(b1.shape), _full(w2.shape), _full(b2.shape)],
        out_specs=pl.BlockSpec((blk, NH), lambda i: (i, 0)),
        out_shape=jax.ShapeDtypeStruct((B * N, NH), jnp.float32),
    )(h, a0, a1, w1, b1, w2, b2)


def _gnn_last_body(h_ref, a0_ref, a1_ref, w1, b1, w2, b2, ow, ob, o_ref):
    z = h_ref[...] + a0_ref[...] + a1_ref[...]
    t = (jnp.maximum(z @ w1[...] + b1[...], 0.0) @ w2[...] + b2[...])
    m = jnp.mean(t, axis=0, keepdims=True)
    o_ref[0] = m @ ow[...] + ob[...]


def _gnn_last(h, a0, a1, w1, b1, w2, b2, ow, ob):
    o3 = pl.pallas_call(
        _gnn_last_body,
        grid=(B,),
        in_specs=[pl.BlockSpec((N, NH), lambda i: (i, 0))] * 3
        + [_full(w.shape) for w in (w1, b1, w2, b2, ow, ob)],
        out_specs=pl.BlockSpec((1, 1, NH), lambda b: (b, 0, 0)),
        out_shape=jax.ShapeDtypeStruct((B, 1, NH), jnp.float32),
    )(h, a0, a1, w1, b1, w2, b2, ow, ob)
    return o3.reshape(B, NH)


_NC, _NS = 2, 16
_NW = _NC * _NS
_ET = B * E
_EW = _ET // _NW
_CH = 128
_NCH = _EW // _CH
_GC = 64
_GNCH = _EW // _GC


def _sc_mesh():
    return plsc.VectorSubcoreMesh(core_axis_name="c", subcore_axis_name="s",
                                  num_cores=_NC, num_subcores=_NS)


def _zero_vmem(buf, nrows):
    zz = jnp.zeros((16,), jnp.float32)

    def row(r, _):
        for c in range(NH // 16):
            buf[r, pl.ds(c * 16, 16)] = zz
        return 0

    lax.fori_loop(0, nrows, row, 0)


def _xgather_body(xpad_hbm, ei_hbm, xs_hbm, xd_hbm, srcf_hbm, dstf_hbm,
                  idx_s, idx_d, rows_s, rows_d, sem_s, sem_d):
    cid = lax.axis_index("c")
    sid = lax.axis_index("s")
    wid = sid * _NC + cid
    base = wid * _EW
    b = wid // (_NW // B)
    r0 = base - b * E
    nr = _EW // _GC
    pltpu.sync_copy(ei_hbm.at[pl.ds(b * 2 * (E // _GC) + r0 // _GC, nr)],
                    idx_s)
    pltpu.sync_copy(
        ei_hbm.at[pl.ds(b * 2 * (E // _GC) + E // _GC + r0 // _GC, nr)],
        idx_d)
    off = jnp.full((16,), b * N, jnp.int32)

    def addoff(r, _):
        for c in range(_GC // 16):
            sl = pl.ds(c * 16, 16)
            idx_s[r, sl] = idx_s[r, sl] + off
            idx_d[r, sl] = idx_d[r, sl] + off
        return 0

    lax.fori_loop(0, nr, addoff, 0)
    cps = []
    for j in range(nr):
        sl = pl.ds(j * _GC, _GC)
        cps.append(pltpu.async_copy(xpad_hbm.at[idx_s.at[j]],
                                    rows_s.at[sl], sem_s))
        cps.append(pltpu.async_copy(xpad_hbm.at[idx_d.at[j]],
                                    rows_d.at[sl], sem_d))
    pltpu.sync_copy(idx_s, srcf_hbm.at[pl.ds(wid * nr, nr)])
    pltpu.sync_copy(idx_d, dstf_hbm.at[pl.ds(wid * nr, nr)])
    for cp in cps:
        cp.wait()
    pltpu.sync_copy(rows_s, xs_hbm.at[pl.ds(base, _EW)])
    pltpu.sync_copy(rows_d, xd_hbm.at[pl.ds(base, _EW)])


def _sc_xgather(xpad, ei_flat):
    f = pl.kernel(
        _xgather_body,
        out_type=[jax.ShapeDtypeStruct((_ET, 16), jnp.float32),
                  jax.ShapeDtypeStruct((_ET, 16), jnp.float32),
                  jax.ShapeDtypeStruct((_ET // _GC, _GC), jnp.int32),
                  jax.ShapeDtypeStruct((_ET // _GC, _GC), jnp.int32)],
        mesh=_sc_mesh(),
        scratch_types=[pltpu.VMEM((_EW // _GC, _GC), jnp.int32),
                       pltpu.VMEM((_EW // _GC, _GC), jnp.int32),
                       pltpu.VMEM((_EW, 16), jnp.float32),
                       pltpu.VMEM((_EW, 16), jnp.float32),
                       pltpu.SemaphoreType.DMA,
                       pltpu.SemaphoreType.DMA],
        compiler_params=pltpu.CompilerParams(use_tc_tiling_on_sc=False),
    )
    return f(xpad, ei_flat)


def _gnnmsg_body(h_hbm, ea_hbm, src_hbm, dst_hbm, out0_hbm, out1_hbm,
                 idx_s, idx_d, rows, eab, zbuf, shared,
                 gsem, esem, ssem):
    cid = lax.axis_index("c")
    sid = lax.axis_index("s")
    wid = sid * _NC + cid
    base = wid * _EW
    stripe = B * N // _NS

    pltpu.sync_copy(src_hbm.at[pl.ds(wid * _GNCH, _GNCH)], idx_s)
    pltpu.sync_copy(dst_hbm.at[pl.ds(wid * _GNCH, _GNCH)], idx_d)
    gcp = [None] * _GNCH
    ecp = [None] * _GNCH
    scp = [None] * _GNCH

    def fire_g(j):
        gcp[j] = pltpu.async_copy(h_hbm.at[idx_s.at[j]],
                                  rows.at[j % 3], gsem[j % 3])

    def fire_e(j):
        ecp[j] = pltpu.async_copy(ea_hbm.at[pl.ds(base + j * _GC, _GC)],
                                  eab.at[j % 2], esem[j % 2])

    fire_g(0), fire_g(1), fire_g(2)
    fire_e(0), fire_e(1)
    _zero_vmem(zbuf, 64)
    for j in range(stripe // 64):
        pltpu.sync_copy(zbuf, shared.at[pl.ds(sid * stripe + j * 64, 64)])
    plsc.subcore_barrier()

    for j in range(_GNCH):
        gcp[j].wait()
        ecp[j].wait()
        rp = rows.at[j % 3]
        ep = eab.at[j % 2]

        @functools.partial(plsc.parallel_loop, 0, _GC, unroll=4)
        def _row(r):
            for c in range(NH // 16):
                sl = pl.ds(c * 16, 16)
                rp[r, sl] = jnp.maximum(rp[r, sl] + ep[r, sl], 0.0)
        scp[j] = pltpu.async_copy(rp, shared.at[idx_d.at[j]],
                                  ssem[j % 3], add=True)
        if j + 2 < _GNCH:
            fire_e(j + 2)
        if j >= 1 and j + 2 < _GNCH:
            scp[j - 1].wait()
            fire_g(j + 2)
    for j in range(max(0, _GNCH - 3), _GNCH):
        scp[j].wait()
    plsc.subcore_barrier()

    src_sl = shared.at[pl.ds(sid * stripe, stripe)]
    out_sl = pl.ds(sid * stripe, stripe)

    @pl.when(cid == 0)
    def _():
        pltpu.sync_copy(src_sl, out0_hbm.at[out_sl])

    @pl.when(cid == 1)
    def _():
        pltpu.sync_copy(src_sl, out1_hbm.at[out_sl])


def _scatter_body(msg_hbm, dst_hbm, out0_hbm, out1_hbm,
                  idx_d, rows, zbuf, shared, msem, ssem):
    cid = lax.axis_index("c")
    sid = lax.axis_index("s")
    wid = sid * _NC + cid
    base = wid * _EW
    stripe = B * N // _NS

    pltpu.sync_copy(dst_hbm.at[pl.ds(wid * _GNCH, _GNCH)], idx_d)
    mcp = [None] * _GNCH
    scp = [None] * _GNCH

    def fire_m(j):
        mcp[j] = pltpu.async_copy(msg_hbm.at[pl.ds(base + j * _GC, _GC)],
                                  rows.at[j % 3], msem[j % 3])

    fire_m(0), fire_m(1), fire_m(2)
    _zero_vmem(zbuf, 64)
    for j in range(stripe // 64):
        pltpu.sync_copy(zbuf, shared.at[pl.ds(sid * stripe + j * 64, 64)])
    plsc.subcore_barrier()

    for j in range(_GNCH):
        mcp[j].wait()
        scp[j] = pltpu.async_copy(rows.at[j % 3], shared.at[idx_d.at[j]],
                                  ssem[j % 3], add=True)
        if j >= 1 and j + 2 < _GNCH:
            scp[j - 1].wait()
            fire_m(j + 2)
    for j in range(max(0, _GNCH - 3), _GNCH):
        scp[j].wait()
    plsc.subcore_barrier()

    src_sl = shared.at[pl.ds(sid * stripe, stripe)]
    out_sl = pl.ds(sid * stripe, stripe)

    @pl.when(cid == 0)
    def _():
        pltpu.sync_copy(src_sl, out0_hbm.at[out_sl])

    @pl.when(cid == 1)
    def _():
        pltpu.sync_copy(src_sl, out1_hbm.at[out_sl])


def _sc_scatter(msg, dst_f):
    f = pl.kernel(
        _scatter_body,
        out_type=[jax.ShapeDtypeStruct((B * N, NH), jnp.float32),
                  jax.ShapeDtypeStruct((B * N, NH), jnp.float32)],
        mesh=_sc_mesh(),
        scratch_types=[pltpu.VMEM((_GNCH, _GC), jnp.int32),
                       pltpu.VMEM((3, _GC, NH), jnp.float32),
                       pltpu.VMEM((64, NH), jnp.float32),
                       pltpu.VMEM_SHARED((B * N, NH), jnp.float32),
                       [pltpu.SemaphoreType.DMA] * 3,
                       [pltpu.SemaphoreType.DMA] * 3],
    )
    return f(msg, dst_f)


def _sc_gnn_msg(h, ea, src_f, dst_f):
    f = pl.kernel(
        _gnnmsg_body,
        out_type=[jax.ShapeDtypeStruct((B * N, NH), jnp.float32),
                  jax.ShapeDtypeStruct((B * N, NH), jnp.float32)],
        mesh=_sc_mesh(),
        scratch_types=[pltpu.VMEM((_GNCH, _GC), jnp.int32),
                       pltpu.VMEM((_GNCH, _GC), jnp.int32),
                       pltpu.VMEM((3, _GC, NH), jnp.float32),
                       pltpu.VMEM((2, _GC, NH), jnp.float32),
                       pltpu.VMEM((64, NH), jnp.float32),
                       pltpu.VMEM_SHARED((B * N, NH), jnp.float32),
                       [pltpu.SemaphoreType.DMA] * 3,
                       [pltpu.SemaphoreType.DMA] * 2,
                       [pltpu.SemaphoreType.DMA] * 3],
    )
    return f(h, ea, src_f, dst_f)


def kernel(x, edge_index, scb, enc1_W1, enc1_b1, enc1_W2, enc1_b2,
           enc2_W1, enc2_b1, enc2_W2, enc2_b2, enc4_W1, enc4_b1,
           enc4_W2, enc4_b2, node_W, node_b, gnn_W1, gnn_b1, gnn_W2,
           gnn_b2, out_W, out_b):
    xpad = jnp.pad(x.reshape(B * N, 2), ((0, 0), (0, 14)))
    xs_g, xd_g, src_f, dst_f = _sc_xgather(xpad,
                                            edge_index.reshape(-1, _GC))

    b2d = lambda v: v.reshape(1, -1)
    edge_attr, h, msg1 = _edge_encoder(
        scb, xs_g.reshape(B, E, 16), xd_g.reshape(B, E, 16), x,
        enc1_W1, b2d(enc1_b1), enc1_W2, b2d(enc1_b2),
        enc2_W1, enc2_b1.reshape(-1, 1),
        enc2_W2, b2d(enc2_b2), enc4_W1, b2d(enc4_b1), enc4_W2, b2d(enc4_b2),
        node_W, b2d(node_b),
    )
    edge_attr = edge_attr.reshape(B * E, NH)
    h = h.reshape(B * N, NH)
    msg1 = msg1.reshape(B * E, NH)

    a0, a1 = _sc_scatter(msg1, dst_f)
    h = _gnn_dense(h, a0, a1, gnn_W1[0], b2d(gnn_b1[0]),
                   gnn_W2[0], b2d(gnn_b2[0]))
    a0, a1 = _sc_gnn_msg(h, edge_attr, src_f, dst_f)
    h = _gnn_dense(h, a0, a1, gnn_W1[1], b2d(gnn_b1[1]),
                   gnn_W2[1], b2d(gnn_b2[1]))
    a0, a1 = _sc_gnn_msg(h, edge_attr, src_f, dst_f)
    return _gnn_last(h, a0, a1, gnn_W1[2], b2d(gnn_b1[2]),
                     gnn_W2[2], b2d(gnn_b2[2]), out_W, b2d(out_b))
